# baseline (speedup 1.0000x reference)
import functools

import jax
import jax.numpy as jnp
from jax import lax
from jax.experimental import pallas as pl
from jax.experimental.pallas import tpu as pltpu
from jax.experimental.pallas import tpu_sc as plsc

VOCAB = 256
SEQ_LEN = 3
BATCH = 4096

NC = 2
NS = 16
L = 16
NW = NC * NS
BPW = BATCH // NW
NV = BPW // L

GRID = 8
SPLIT = 224
TAIL0 = SPLIT * VOCAB
RPW = (VOCAB * VOCAB - TAIL0) // NW
CH = 32
NG = RPW // CH
SEGS = VOCAB // L


@functools.partial(
    pl.kernel,
    mesh=plsc.VectorSubcoreMesh(core_axis_name="c", subcore_axis_name="s"),
    out_type=(jax.ShapeDtypeStruct((NW, L), jnp.float32),
              jax.ShapeDtypeStruct((NW, L), jnp.float32)),
    scratch_types=[
        pltpu.VMEM((SEQ_LEN, BPW), jnp.int32),
        pltpu.VMEM((BPW,), jnp.int32),
        pltpu.VMEM((BPW, VOCAB), jnp.float32),
        pltpu.VMEM((CH, VOCAB), jnp.float32),
        pltpu.VMEM((CH, VOCAB), jnp.float32),
        pltpu.VMEM((1, L), jnp.float32),
        pltpu.VMEM((1, L), jnp.float32),
        pltpu.SemaphoreType.DMA,
        pltpu.SemaphoreType.DMA,
        pltpu.SemaphoreType.DMA,
    ],
)
def _sc_gather(tcols_hbm, wrows_hbm, out_hbm, oute_hbm, t_v, q_v, rows_v,
               ch0_v, ch1_v, acc_v, acce_v, sem, sem0, sem1):
    wid = lax.axis_index("s") * NC + lax.axis_index("c")
    base = wid * BPW
    pltpu.sync_copy(tcols_hbm.at[:, pl.ds(base, BPW)], t_v)
    iota = lax.iota(jnp.int32, L)

    def _qbody(j, carry):
        sl = pl.ds(pl.multiple_of(j * L, L), L)
        t0 = t_v[0, sl]
        t1 = t_v[1, sl]
        q_v[sl] = t0 * VOCAB + t1
        return carry

    lax.fori_loop(0, NV, _qbody, 0)
    gat = pltpu.async_copy(wrows_hbm.at[q_v], rows_v, sem)

    tbase = TAIL0 + wid * RPW
    bufs = (ch0_v, ch1_v)
    sems = (sem0, sem1)
    descs = [None, None]
    descs[0] = pltpu.async_copy(
        wrows_hbm.at[pl.ds(tbase, CH), :], ch0_v, sem0)

    def _ebody(buf):
        def _erow(r, acc):
            for s in range(SEGS):
                acc = acc + jnp.exp(buf[r, pl.ds(s * L, L)])
            return acc
        return _erow

    acc_e = jnp.zeros((L,), jnp.float32)
    for g in range(NG):
        if g + 1 < NG:
            nxt = (g + 1) % 2
            descs[nxt] = pltpu.async_copy(
                wrows_hbm.at[pl.ds(tbase + (g + 1) * CH, CH), :],
                bufs[nxt], sems[nxt])
        descs[g % 2].wait()
        acc_e = lax.fori_loop(0, CH, _ebody(bufs[g % 2]), acc_e)
    acce_v[0, :] = acc_e
    pltpu.sync_copy(acce_v, oute_hbm.at[pl.ds(wid, 1), :])

    gat.wait()

    def _sbody(j, acc):
        sl = pl.ds(pl.multiple_of(j * L, L), L)
        t2g = t_v[2, sl]
        gvec = lax.shift_right_logical(t2g, 4) * L
        lanevec = lax.bitwise_and(t2g, L - 1)
        for i in range(L):
            seg = rows_v[j * L + i, pl.ds(pl.multiple_of(gvec[i], L), L)]
            acc = acc + jnp.where(iota == lanevec[i], seg, 0.0)
        return acc

    acc = lax.fori_loop(0, NV, _sbody, jnp.zeros((L,), jnp.float32))
    acc_v[0, :] = acc
    pltpu.sync_copy(acc_v, out_hbm.at[pl.ds(wid, 1), :])


def _tc_body(w_ref, out_ref):
    i = pl.program_id(0)

    @pl.when(i == 0)
    def _init():
        out_ref[0, 0] = 0.0

    out_ref[0, 0] += jnp.sum(jnp.exp(w_ref[...]))


def kernel(target, W):
    tcols = target.astype(jnp.int32).T
    wrows = W.reshape(VOCAB * VOCAB, VOCAB)
    partials, exp_partials = _sc_gather(tcols, wrows)
    tc_sum = pl.pallas_call(
        _tc_body,
        grid=(GRID,),
        in_specs=[
            pl.BlockSpec((SPLIT // GRID, VOCAB, VOCAB), lambda i: (i, 0, 0)),
        ],
        out_specs=pl.BlockSpec(memory_space=pltpu.SMEM),
        out_shape=jax.ShapeDtypeStruct((1, 1), jnp.float32),
    )(W)
    z = tc_sum[0, 0] + jnp.sum(exp_partials)
    return jnp.log(z) - jnp.sum(partials) * (1.0 / BATCH)


# speedup vs baseline: 1.0371x; 1.0371x over previous
import functools

import jax
import jax.numpy as jnp
from jax import lax
from jax.experimental import pallas as pl
from jax.experimental.pallas import tpu as pltpu
from jax.experimental.pallas import tpu_sc as plsc

VOCAB = 256
SEQ_LEN = 3
BATCH = 4096

NC = 2
NS = 16
L = 16
NW = NC * NS
BPW = BATCH // NW
NV = BPW // L
ROWS = VOCAB ** SEQ_LEN // L

GRID = 4
BLK_ROWS = VOCAB * VOCAB * VOCAB // VOCAB // GRID


@functools.partial(
    pl.kernel,
    mesh=plsc.VectorSubcoreMesh(core_axis_name="c", subcore_axis_name="s"),
    out_type=jax.ShapeDtypeStruct((NW, L), jnp.float32),
    scratch_types=[
        pltpu.VMEM((SEQ_LEN, BPW), jnp.int32),
        pltpu.VMEM((BPW,), jnp.int32),
        pltpu.VMEM((BPW, VOCAB), jnp.float32),
        pltpu.VMEM((1, L), jnp.float32),
        pltpu.SemaphoreType.DMA,
    ],
)
def _sc_gather(tcols_hbm, wrows_hbm, out_hbm, t_v, q_v, rows_v,
               acc_v, sem):
    wid = lax.axis_index("s") * NC + lax.axis_index("c")
    base = wid * BPW
    pltpu.sync_copy(tcols_hbm.at[:, pl.ds(base, BPW)], t_v)
    iota = lax.iota(jnp.int32, L)

    def _qbody(j, carry):
        sl = pl.ds(pl.multiple_of(j * L, L), L)
        t0 = t_v[0, sl]
        t1 = t_v[1, sl]
        t2 = t_v[2, sl]
        q_v[sl] = t0 * VOCAB + t1
        return carry

    lax.fori_loop(0, NV, _qbody, 0)
    pltpu.async_copy(wrows_hbm.at[q_v], rows_v, sem).wait()

    def _sbody(j, acc):
        sl = pl.ds(pl.multiple_of(j * L, L), L)
        t2g = t_v[2, sl]
        gvec = lax.shift_right_logical(t2g, 4) * L
        lanevec = lax.bitwise_and(t2g, L - 1)
        for i in range(L):
            seg = rows_v[j * L + i, pl.ds(pl.multiple_of(gvec[i], L), L)]
            acc = acc + jnp.where(iota == lanevec[i], seg, 0.0)
        return acc

    acc = lax.fori_loop(0, NV, _sbody, jnp.zeros((L,), jnp.float32))
    acc_v[0, :] = acc
    pltpu.sync_copy(acc_v, out_hbm.at[pl.ds(wid, 1), :])


def _tc_body(w_ref, out_ref):
    i = pl.program_id(0)

    @pl.when(i == 0)
    def _init():
        out_ref[0, 0] = 0.0

    out_ref[0, 0] += jnp.sum(jnp.exp(w_ref[...]))

    @pl.when(i == pl.num_programs(0) - 1)
    def _finish():
        out_ref[0, 0] = jnp.log(out_ref[0, 0])


def kernel(target, W):
    tcols = target.astype(jnp.int32).T
    wrows = W.reshape(VOCAB * VOCAB, VOCAB)
    log_z = pl.pallas_call(
        _tc_body,
        grid=(GRID,),
        in_specs=[
            pl.BlockSpec((VOCAB // GRID, VOCAB, VOCAB), lambda i: (i, 0, 0)),
        ],
        out_specs=pl.BlockSpec(memory_space=pltpu.SMEM),
        out_shape=jax.ShapeDtypeStruct((1, 1), jnp.float32),
    )(W)
    partials = _sc_gather(tcols, wrows)
    return log_z[0, 0] - jnp.sum(partials) * (1.0 / BATCH)


# speedup vs baseline: 1.0414x; 1.0041x over previous
import functools

import jax
import jax.numpy as jnp
from jax import lax
from jax.experimental import pallas as pl
from jax.experimental.pallas import tpu as pltpu
from jax.experimental.pallas import tpu_sc as plsc

VOCAB = 256
SEQ_LEN = 3
BATCH = 4096

NC = 2
NS = 16
L = 16
NW = NC * NS
BPW = BATCH // NW
NV = BPW // L

GRID = 8


@functools.partial(
    pl.kernel,
    mesh=plsc.VectorSubcoreMesh(core_axis_name="c", subcore_axis_name="s"),
    out_type=jax.ShapeDtypeStruct((NW, L), jnp.float32),
    scratch_types=[
        pltpu.VMEM((SEQ_LEN, BPW), jnp.int32),
        pltpu.VMEM((BPW,), jnp.int32),
        pltpu.VMEM((BPW, VOCAB), jnp.float32),
        pltpu.VMEM((1, L), jnp.float32),
        pltpu.SemaphoreType.DMA,
    ],
)
def _sc_gather(tcols_hbm, wrows_hbm, out_hbm, t_v, q_v, rows_v,
               acc_v, sem):
    wid = lax.axis_index("s") * NC + lax.axis_index("c")
    base = wid * BPW
    pltpu.sync_copy(tcols_hbm.at[:, pl.ds(base, BPW)], t_v)
    iota = lax.iota(jnp.int32, L)

    def _qbody(j, carry):
        sl = pl.ds(pl.multiple_of(j * L, L), L)
        t0 = t_v[0, sl]
        t1 = t_v[1, sl]
        t2 = t_v[2, sl]
        q_v[sl] = t0 * VOCAB + t1
        return carry

    lax.fori_loop(0, NV, _qbody, 0)
    pltpu.async_copy(wrows_hbm.at[q_v], rows_v, sem).wait()

    def _sbody(j, acc):
        sl = pl.ds(pl.multiple_of(j * L, L), L)
        t2g = t_v[2, sl]
        gvec = lax.shift_right_logical(t2g, 4) * L
        lanevec = lax.bitwise_and(t2g, L - 1)
        for i in range(L):
            seg = rows_v[j * L + i, pl.ds(pl.multiple_of(gvec[i], L), L)]
            acc = acc + jnp.where(iota == lanevec[i], seg, 0.0)
        return acc

    acc = lax.fori_loop(0, NV, _sbody, jnp.zeros((L,), jnp.float32))
    acc_v[0, :] = acc
    pltpu.sync_copy(acc_v, out_hbm.at[pl.ds(wid, 1), :])


def _tc_body(w_ref, out_ref):
    i = pl.program_id(0)

    @pl.when(i == 0)
    def _init():
        out_ref[0, 0] = 0.0

    out_ref[0, 0] += jnp.sum(jnp.exp(w_ref[...]))

    @pl.when(i == pl.num_programs(0) - 1)
    def _finish():
        out_ref[0, 0] = jnp.log(out_ref[0, 0])


def kernel(target, W):
    tcols = target.astype(jnp.int32).T
    wrows = W.reshape(VOCAB * VOCAB, VOCAB)
    log_z = pl.pallas_call(
        _tc_body,
        grid=(GRID,),
        in_specs=[
            pl.BlockSpec((VOCAB // GRID, VOCAB, VOCAB), lambda i: (i, 0, 0)),
        ],
        out_specs=pl.BlockSpec(memory_space=pltpu.SMEM),
        out_shape=jax.ShapeDtypeStruct((1, 1), jnp.float32),
    )(W)
    partials = _sc_gather(tcols, wrows)
    return log_z[0, 0] - jnp.sum(partials) * (1.0 / BATCH)


# speedup vs baseline: 1.0434x; 1.0020x over previous
import functools

import jax
import jax.numpy as jnp
from jax import lax
from jax.experimental import pallas as pl
from jax.experimental.pallas import tpu as pltpu
from jax.experimental.pallas import tpu_sc as plsc

VOCAB = 256
SEQ_LEN = 3
BATCH = 4096

NC = 2
NS = 16
L = 16
NW = NC * NS
BPW = BATCH // NW
NV = BPW // L

GRID = 8


@functools.partial(
    pl.kernel,
    mesh=plsc.VectorSubcoreMesh(core_axis_name="c", subcore_axis_name="s"),
    out_type=jax.ShapeDtypeStruct((NW, L), jnp.float32),
    scratch_types=[
        pltpu.VMEM((SEQ_LEN, BPW), jnp.int32),
        pltpu.VMEM((BPW,), jnp.int32),
        pltpu.VMEM((BPW, VOCAB), jnp.float32),
        pltpu.VMEM((1, L), jnp.float32),
        pltpu.SemaphoreType.DMA,
    ],
)
def _sc_gather(tcols_hbm, wrows_hbm, out_hbm, t_v, q_v, rows_v,
               acc_v, sem):
    wid = lax.axis_index("s") * NC + lax.axis_index("c")
    base = wid * BPW
    pltpu.sync_copy(tcols_hbm.at[:, pl.ds(base, BPW)], t_v)
    iota = lax.iota(jnp.int32, L)

    def _qbody(j, carry):
        sl = pl.ds(pl.multiple_of(j * L, L), L)
        t0 = t_v[0, sl]
        t1 = t_v[1, sl]
        q_v[sl] = t0 * VOCAB + t1
        return carry

    lax.fori_loop(0, NV, _qbody, 0)
    pltpu.async_copy(wrows_hbm.at[q_v], rows_v, sem).wait()

    def _sbody(j, acc):
        sl = pl.ds(pl.multiple_of(j * L, L), L)
        t2g = t_v[2, sl]
        gvec = lax.shift_right_logical(t2g, 4) * L
        lanevec = lax.bitwise_and(t2g, L - 1)
        for i in range(L):
            seg = rows_v[j * L + i, pl.ds(pl.multiple_of(gvec[i], L), L)]
            acc = acc + jnp.where(iota == lanevec[i], seg, 0.0)
        return acc

    acc = lax.fori_loop(0, NV, _sbody, jnp.zeros((L,), jnp.float32))
    acc_v[0, :] = acc
    pltpu.sync_copy(acc_v, out_hbm.at[pl.ds(wid, 1), :])


def _tc_body(w_ref, out_ref):
    i = pl.program_id(0)

    @pl.when(i == 0)
    def _init():
        out_ref[0, 0] = 0.0

    out_ref[0, 0] += jnp.sum(jnp.exp(w_ref[...]))

    @pl.when(i == pl.num_programs(0) - 1)
    def _finish():
        out_ref[0, 0] = jnp.log(out_ref[0, 0])


def kernel(target, W):
    tcols = target.astype(jnp.int32).T
    wrows = W.reshape(VOCAB * VOCAB, VOCAB)
    log_z = pl.pallas_call(
        _tc_body,
        grid=(GRID,),
        in_specs=[
            pl.BlockSpec((VOCAB // GRID, VOCAB, VOCAB), lambda i: (i, 0, 0)),
        ],
        out_specs=pl.BlockSpec(memory_space=pltpu.SMEM),
        out_shape=jax.ShapeDtypeStruct((1, 1), jnp.float32),
    )(W)
    partials = _sc_gather(tcols, wrows)
    return log_z[0, 0] - jnp.sum(partials) * (1.0 / BATCH)
